# baseline (speedup 1.0000x reference)
import functools

import jax
import jax.numpy as jnp
from jax import lax
from jax.experimental import pallas as pl
from jax.experimental.pallas import tpu as pltpu
from jax.experimental.pallas import tpu_sc as plsc

_L = 16


def kernel(x, pos_table):
    B, S, D = x.shape
    NC, NS = 2, 16
    NW = NC * NS
    SW = S // NW
    R = 8
    NCH = SW // R
    CW = R * D

    xf = x.reshape(B * S * D)
    tf = pos_table.reshape(S * D)
    mesh = plsc.VectorSubcoreMesh(core_axis_name="c", subcore_axis_name="s")

    @functools.partial(
        pl.kernel,
        out_type=jax.ShapeDtypeStruct((B * S * D,), jnp.float32),
        mesh=mesh,
        scratch_types=[
            pltpu.VMEM((2, CW), jnp.float32),
            pltpu.VMEM((2, B, CW), jnp.float32),
            pltpu.SemaphoreType.DMA,
            pltpu.SemaphoreType.DMA,
            pltpu.SemaphoreType.DMA,
            pltpu.SemaphoreType.DMA,
        ],
    )
    def body(x_hbm, t_hbm, o_hbm, tbuf, xbuf, sin0, sin1, sout0, sout1):
        wid = lax.axis_index("s") * NC + lax.axis_index("c")
        s0 = wid * SW
        sin = (sin0, sin1)
        sout = (sout0, sout1)

        def start_in(c, slot):
            tb = (s0 + c * R) * D
            hs = [pltpu.async_copy(t_hbm.at[pl.ds(tb, CW)], tbuf.at[slot], sin[slot])]
            for b in range(B):
                xb = (b * S + s0 + c * R) * D
                hs.append(
                    pltpu.async_copy(x_hbm.at[pl.ds(xb, CW)], xbuf.at[slot, b], sin[slot])
                )
            return hs

        def start_out(c, slot):
            hs = []
            for b in range(B):
                ob = (b * S + s0 + c * R) * D
                hs.append(
                    pltpu.async_copy(xbuf.at[slot, b], o_hbm.at[pl.ds(ob, CW)], sout[slot])
                )
            return hs

        def compute(slot):
            def step(i, carry):
                off = i * _L
                t = tbuf[slot, pl.ds(off, _L)]
                for b in range(B):
                    plsc.addupdate(xbuf.at[slot, b, pl.ds(off, _L)], t)
                return carry

            lax.fori_loop(0, CW // _L, step, 0)

        ins = [start_in(0, 0), start_in(1, 1)]
        outs = [None, None]
        for c in range(NCH):
            slot = c % 2
            for h in ins[slot]:
                h.wait()
            compute(slot)
            outs[slot] = start_out(c, slot)
            if c + 2 < NCH:
                for h in outs[slot]:
                    h.wait()
                ins[slot] = start_in(c + 2, slot)
        for c in (NCH - 2, NCH - 1):
            for h in outs[c % 2]:
                h.wait()

    out = body(xf, tf)
    return out.reshape(B, S, D)


# speedup vs baseline: 2.9426x; 2.9426x over previous
import functools

import jax
import jax.numpy as jnp
from jax import lax
from jax.experimental import pallas as pl
from jax.experimental.pallas import tpu as pltpu
from jax.experimental.pallas import tpu_sc as plsc

_L = 16


def kernel(x, pos_table):
    B, S, D = x.shape
    NC, NS = 2, 16
    NW = NC * NS
    SW = S // NW
    R = 8
    NCH = SW // R

    x2 = x.reshape(B * S, D)
    mesh = plsc.VectorSubcoreMesh(core_axis_name="c", subcore_axis_name="s")

    @functools.partial(
        pl.kernel,
        out_type=jax.ShapeDtypeStruct((B * S, D), jnp.float32),
        mesh=mesh,
        scratch_types=[
            pltpu.VMEM((2, R, D), jnp.float32),
            pltpu.VMEM((2, B, R, D), jnp.float32),
            pltpu.SemaphoreType.DMA,
            pltpu.SemaphoreType.DMA,
            pltpu.SemaphoreType.DMA,
            pltpu.SemaphoreType.DMA,
        ],
        compiler_params=pltpu.CompilerParams(use_tc_tiling_on_sc=True),
    )
    def body(x_hbm, t_hbm, o_hbm, tbuf, xbuf, sin0, sin1, sout0, sout1):
        wid = lax.axis_index("s") * NC + lax.axis_index("c")
        s0 = wid * SW
        sin = (sin0, sin1)
        sout = (sout0, sout1)

        def start_in(c, slot):
            tr = s0 + c * R
            hs = [pltpu.async_copy(t_hbm.at[pl.ds(tr, R), :], tbuf.at[slot], sin[slot])]
            for b in range(B):
                xr = b * S + s0 + c * R
                hs.append(
                    pltpu.async_copy(x_hbm.at[pl.ds(xr, R), :], xbuf.at[slot, b], sin[slot])
                )
            return hs

        def start_out(c, slot):
            hs = []
            for b in range(B):
                orow = b * S + s0 + c * R
                hs.append(
                    pltpu.async_copy(xbuf.at[slot, b], o_hbm.at[pl.ds(orow, R), :], sout[slot])
                )
            return hs

        def compute(slot):
            def step(j, carry):
                off = j * _L
                for r in range(R):
                    t = tbuf[slot, r, pl.ds(off, _L)]
                    for b in range(B):
                        plsc.addupdate(xbuf.at[slot, b, r, pl.ds(off, _L)], t)
                return carry

            lax.fori_loop(0, D // _L, step, 0)

        ins = [start_in(0, 0), start_in(1, 1)]
        outs = [None, None]
        for c in range(NCH):
            slot = c % 2
            for h in ins[slot]:
                h.wait()
            compute(slot)
            outs[slot] = start_out(c, slot)
            if c + 2 < NCH:
                for h in outs[slot]:
                    h.wait()
                ins[slot] = start_in(c + 2, slot)
        for c in (NCH - 2, NCH - 1):
            for h in outs[c % 2]:
                h.wait()

    out = body(x2, pos_table)
    return out.reshape(B, S, D)


# speedup vs baseline: 3.3754x; 1.1471x over previous
import functools

import jax
import jax.numpy as jnp
from jax import lax
from jax.experimental import pallas as pl
from jax.experimental.pallas import tpu as pltpu
from jax.experimental.pallas import tpu_sc as plsc

_L = 16


def kernel(x, pos_table):
    B, S, D = x.shape
    NC, NS = 2, 16
    NW = NC * NS
    SW = S // NW
    R = 8
    NCH = SW // R

    x2 = x.reshape(B * S, D)
    mesh = plsc.VectorSubcoreMesh(core_axis_name="c", subcore_axis_name="s")

    @functools.partial(
        pl.kernel,
        out_type=jax.ShapeDtypeStruct((B * S, D), jnp.float32),
        mesh=mesh,
        scratch_types=[
            pltpu.VMEM((2, R, D), jnp.float32),
            pltpu.VMEM((3, B, R, D), jnp.float32),
            pltpu.SemaphoreType.DMA,
            pltpu.SemaphoreType.DMA,
            pltpu.SemaphoreType.DMA,
            pltpu.SemaphoreType.DMA,
            pltpu.SemaphoreType.DMA,
            pltpu.SemaphoreType.DMA,
            pltpu.SemaphoreType.DMA,
            pltpu.SemaphoreType.DMA,
        ],
        compiler_params=pltpu.CompilerParams(use_tc_tiling_on_sc=True),
    )
    def body(x_hbm, t_hbm, o_hbm, tbuf, xbuf,
             sin0, sin1, sin2, sout0, sout1, sout2, st0, st1):
        wid = lax.axis_index("s") * NC + lax.axis_index("c")
        s0 = wid * SW
        sin = (sin0, sin1, sin2)
        sout = (sout0, sout1, sout2)
        st = (st0, st1)

        def start_tin(c):
            tr = s0 + c * R
            return pltpu.async_copy(t_hbm.at[pl.ds(tr, R), :], tbuf.at[c % 2], st[c % 2])

        def start_in(c):
            slot = c % 3
            hs = []
            for b in range(B):
                xr = b * S + s0 + c * R
                hs.append(
                    pltpu.async_copy(x_hbm.at[pl.ds(xr, R), :], xbuf.at[slot, b], sin[slot])
                )
            return hs

        def start_out(c):
            slot = c % 3
            hs = []
            for b in range(B):
                orow = b * S + s0 + c * R
                hs.append(
                    pltpu.async_copy(xbuf.at[slot, b], o_hbm.at[pl.ds(orow, R), :], sout[slot])
                )
            return hs

        def compute(c):
            slot = c % 3
            tslot = c % 2

            def step(j, carry):
                off = j * _L
                for r in range(R):
                    t = tbuf[tslot, r, pl.ds(off, _L)]
                    for b in range(B):
                        plsc.addupdate(xbuf.at[slot, b, r, pl.ds(off, _L)], t)
                return carry

            lax.fori_loop(0, D // _L, step, 0)

        tins = {0: start_tin(0), 1: start_tin(1)}
        ins = {0: start_in(0), 1: start_in(1), 2: start_in(2)}
        outs = {}
        outs_waited = set()
        for c in range(NCH):
            for h in ins[c]:
                h.wait()
            tins[c].wait()
            compute(c)
            if c + 2 < NCH:
                tins[c + 2] = start_tin(c + 2)
            outs[c] = start_out(c)
            if c >= 1 and c + 2 < NCH:
                for h in outs[c - 1]:
                    h.wait()
                outs_waited.add(c - 1)
                ins[c + 2] = start_in(c + 2)
        for c in range(NCH):
            if c not in outs_waited:
                for h in outs[c]:
                    h.wait()

    out = body(x2, pos_table)
    return out.reshape(B, S, D)
